# baseline (speedup 1.0000x reference)
import functools

import jax
import jax.numpy as jnp
from jax import lax
from jax.experimental import pallas as pl
from jax.experimental.pallas import tpu as pltpu
from jax.experimental.pallas import tpu_sc as plsc

_NUM_CLASS = 19
_MARGIN = 0.5
_FACTOR = 1.0

_NC = 2
_NS = 16
_NW = _NC * _NS
_L = 16

_C = 96
_H = 512
_W = 512
_RPW = _H // _NS
_PPW = _RPW * _W
_KPAD = _NUM_CLASS * _L
_ACC = _C * _KPAD
_JV = _W // _L


_CG = 4
_NG = _C // _CG
_RH = _RPW // 2


def _sc_body(feat_hbm, lab_hbm, psum_hbm, pcnt_hbm,
             lbuf, fbuf0, fbuf1, fbuf2, fbuf3, acc, cnt,
             sem0, sem1, sem2, sem3):
    wid = lax.axis_index("c") * _NS + lax.axis_index("s")
    b = wid // _NS
    h0 = (wid % _NS) * _RPW

    iota = lax.iota(jnp.int32, _L)
    ones = jnp.ones((_L,), jnp.float32)
    zeros = jnp.zeros((_L,), jnp.float32)

    pltpu.sync_copy(lab_hbm.at[pl.ds(b * _H + h0, _RPW), :], lbuf)

    def _feat_src(c):
        row = (b * _C + c) * _H + h0
        return feat_hbm.at[pl.ds(row, _RPW), :]

    pltpu.make_async_copy(_feat_src(0), fbuf0, sem0).start()
    pltpu.make_async_copy(_feat_src(1), fbuf1, sem1).start()
    pltpu.make_async_copy(_feat_src(2), fbuf2, sem2).start()
    pltpu.make_async_copy(_feat_src(3), fbuf3, sem3).start()

    @plsc.parallel_loop(0, _ACC // _L, unroll=8)
    def _zacc(i):
        acc[pl.ds(i * _L, _L)] = zeros

    @plsc.parallel_loop(0, _KPAD // _L, unroll=1)
    def _zcnt(i):
        cnt[pl.ds(i * _L, _L)] = zeros

    @plsc.parallel_loop(0, _RPW, unroll=2)
    def _cbody(i):
        for j in range(_JV):
            lv = lbuf[i, pl.ds(j * _L, _L)]
            plsc.addupdate_scatter(cnt, [lv * _L + iota], ones)

    bufs = ((fbuf0, sem0), (fbuf1, sem1), (fbuf2, sem2), (fbuf3, sem3))

    def _chan(cb, carry):
        for par, (fb, sem) in enumerate(bufs):
            c = cb * 4 + par
            pltpu.make_async_copy(_feat_src(0), fb, sem).wait()
            base = iota + c * _KPAD

            @plsc.parallel_loop(0, _RPW * _JV, unroll=8)
            def _inner(t):
                i = t // _JV
                j = t % _JV
                lv = lbuf[i, pl.ds(j * _L, _L)]
                v = fb[i, pl.ds(j * _L, _L)]
                plsc.addupdate_scatter(acc, [lv * _L + base], v)

            @pl.when(cb < _C // 4 - 1)
            def _prefetch():
                pltpu.make_async_copy(_feat_src(c + 4), fb, sem).start()
        return carry

    lax.fori_loop(0, _C // 4, _chan, 0)

    pltpu.sync_copy(acc, psum_hbm.at[pl.ds(wid * _ACC, _ACC)])
    pltpu.sync_copy(cnt, pcnt_hbm.at[pl.ds(wid * _KPAD, _KPAD)])


_sc_segsum = functools.partial(
    pl.kernel,
    out_type=(
        jax.ShapeDtypeStruct((_NW * _ACC,), jnp.float32),
        jax.ShapeDtypeStruct((_NW * _KPAD,), jnp.float32),
    ),
    mesh=plsc.VectorSubcoreMesh(
        core_axis_name="c", subcore_axis_name="s",
        num_cores=_NC, num_subcores=_NS),
    compiler_params=pltpu.CompilerParams(
        needs_layout_passes=False, use_tc_tiling_on_sc=True),
    scratch_types=[
        pltpu.VMEM((_RPW, _W), jnp.int32),
        pltpu.VMEM((_RPW, _W), jnp.float32),
        pltpu.VMEM((_RPW, _W), jnp.float32),
        pltpu.VMEM((_RPW, _W), jnp.float32),
        pltpu.VMEM((_RPW, _W), jnp.float32),
        pltpu.VMEM((_ACC,), jnp.float32),
        pltpu.VMEM((_KPAD,), jnp.float32),
        pltpu.SemaphoreType.DMA,
        pltpu.SemaphoreType.DMA,
        pltpu.SemaphoreType.DMA,
        pltpu.SemaphoreType.DMA,
    ],
)(_sc_body)


def _loss_body(psum_ref, pcnt_ref, out_ref):
    s4 = psum_ref[...]
    c3 = pcnt_ref[...]
    s = jnp.sum(jnp.sum(s4, axis=3), axis=0)
    cnt = jnp.sum(jnp.sum(c3, axis=2), axis=0, keepdims=True)
    m = jnp.where(cnt > 0.0, s / jnp.maximum(cnt, 1.0), 0.0)
    n2 = jnp.sum(m * m, axis=0, keepdims=True)
    fn = m / jnp.maximum(jnp.sqrt(n2), 1e-12)
    g = lax.dot_general(fn, fn, (((0,), (0,)), ((), ())),
                        preferred_element_type=jnp.float32)
    d = 1.0 - g
    ii = lax.broadcasted_iota(jnp.int32, (_NUM_CLASS, _NUM_CLASS), 0)
    jj = lax.broadcasted_iota(jnp.int32, (_NUM_CLASS, _NUM_CLASS), 1)
    d = jnp.where(ii == jj, 2.0, d)
    presentf = jnp.where(cnt > 0.0, 1.0, 0.0)
    pair = lax.dot_general(presentf, presentf, (((0,), (0,)), ((), ())),
                           preferred_element_type=jnp.float32)
    vals = pair * jnp.maximum(_MARGIN - d, 0.0)
    n = jnp.sum(presentf)
    out_ref[...] = jnp.reshape(_FACTOR * jnp.sum(vals) / (n * n), (1, 1))


_loss_tc = pl.pallas_call(
    _loss_body,
    out_shape=jax.ShapeDtypeStruct((1, 1), jnp.float32),
)


def kernel(features, labels, prototypes):
    del prototypes
    feat2 = features.reshape(2 * _C * _H, _W)
    lab2 = labels.reshape(2 * _H, _W)
    psum, pcnt = _sc_segsum(feat2, lab2)
    loss = _loss_tc(psum.reshape(_NW, _C, _NUM_CLASS, _L),
                    pcnt.reshape(_NW, _NUM_CLASS, _L))
    return loss[0, 0]


# speedup vs baseline: 1.1930x; 1.1930x over previous
import functools

import jax
import jax.numpy as jnp
from jax import lax
from jax.experimental import pallas as pl
from jax.experimental.pallas import tpu as pltpu
from jax.experimental.pallas import tpu_sc as plsc

_NUM_CLASS = 19
_MARGIN = 0.5
_FACTOR = 1.0

_NC = 2
_NS = 16
_NW = _NC * _NS
_L = 16

_C = 96
_H = 512
_W = 512
_RPW = _H // _NS
_PPW = _RPW * _W
_KPAD = 512
_ACCP = _C * _KPAD
_CNTP = 1024
_JV = _W // _L


_CG = 4
_NG = _C // _CG
_RH = _RPW // 2


def _sc_body(feat_hbm, lab_hbm, psum_hbm, pcnt_hbm,
             lbuf, fbuf0, fbuf1, acc, cnt, sem0, sem1):
    wid = lax.axis_index("c") * _NS + lax.axis_index("s")
    b = wid // _NS
    h0 = (wid % _NS) * _RPW

    iota = lax.iota(jnp.int32, _L)
    ones = jnp.ones((_L,), jnp.float32)
    zeros = jnp.zeros((_L,), jnp.float32)

    pltpu.sync_copy(lab_hbm.at[pl.ds(b * _H + h0, _RPW), :], lbuf)

    def _feat_src(c):
        row = (b * _C + c) * _H + h0
        return feat_hbm.at[pl.ds(row, _RPW), :]

    pltpu.make_async_copy(_feat_src(0), fbuf0, sem0).start()
    pltpu.make_async_copy(_feat_src(1), fbuf1, sem1).start()

    @plsc.parallel_loop(0, _ACCP // _L, unroll=8)
    def _zacc(i):
        acc[pl.ds(i * _L, _L)] = zeros

    @plsc.parallel_loop(0, _CNTP // _L, unroll=4)
    def _zcnt(i):
        cnt[pl.ds(i * _L, _L)] = zeros

    @plsc.parallel_loop(0, _RPW, unroll=2)
    def _cbody(i):
        for j in range(_JV):
            lv = lbuf[i, pl.ds(j * _L, _L)]
            plsc.addupdate_scatter(cnt, [lv * _L + iota], ones)

    bufs = ((fbuf0, sem0), (fbuf1, sem1))

    def _chan(cb, carry):
        for par, (fb, sem) in enumerate(bufs):
            c = cb * 2 + par
            pltpu.make_async_copy(_feat_src(0), fb, sem).wait()
            base = iota + c * _KPAD

            @plsc.parallel_loop(0, _RPW * _JV, unroll=8)
            def _inner(t):
                i = t // _JV
                j = t % _JV
                lv = lbuf[i, pl.ds(j * _L, _L)]
                v = fb[i, pl.ds(j * _L, _L)]
                plsc.addupdate_scatter(acc, [lv * _L + base], v)

            @pl.when(cb < _C // 2 - 1)
            def _prefetch():
                pltpu.make_async_copy(_feat_src(c + 2), fb, sem).start()
        return carry

    lax.fori_loop(0, _C // 2, _chan, 0)

    pltpu.sync_copy(acc, psum_hbm.at[pl.ds(wid * _ACCP, _ACCP)])
    pltpu.sync_copy(cnt, pcnt_hbm.at[pl.ds(wid * _CNTP, _CNTP)])


_sc_segsum = functools.partial(
    pl.kernel,
    out_type=(
        jax.ShapeDtypeStruct((_NW * _ACCP,), jnp.float32),
        jax.ShapeDtypeStruct((_NW * _CNTP,), jnp.float32),
    ),
    mesh=plsc.VectorSubcoreMesh(
        core_axis_name="c", subcore_axis_name="s",
        num_cores=_NC, num_subcores=_NS),
    compiler_params=pltpu.CompilerParams(
        needs_layout_passes=False, use_tc_tiling_on_sc=True),
    scratch_types=[
        pltpu.VMEM((_RPW, _W), jnp.int32),
        pltpu.VMEM((_RPW, _W), jnp.float32),
        pltpu.VMEM((_RPW, _W), jnp.float32),
        pltpu.VMEM((_ACCP,), jnp.float32),
        pltpu.VMEM((_CNTP,), jnp.float32),
        pltpu.SemaphoreType.DMA,
        pltpu.SemaphoreType.DMA,
    ],
)(_sc_body)


def _loss_body(psum_ref, pcnt_ref, out_ref):
    x = psum_ref[...].reshape(_NW, _ACCP // 128, 128)
    z = jnp.sum(x, axis=0)
    ii = lax.broadcasted_iota(jnp.int32, (128, 8), 0)
    jj = lax.broadcasted_iota(jnp.int32, (128, 8), 1)
    sel = jnp.where(ii // _L == jj, 1.0, 0.0)
    y = lax.dot_general(z, sel, (((1,), (0,)), ((), ())),
                        preferred_element_type=jnp.float32)
    ri = lax.broadcasted_iota(jnp.int32, (_C, 4 * _C), 0)
    rj = lax.broadcasted_iota(jnp.int32, (_C, 4 * _C), 1)
    parts = []
    for a in range(3):
        ga = jnp.where(rj == 4 * ri + a, 1.0, 0.0)
        parts.append(lax.dot_general(ga, y, (((1,), (0,)), ((), ())),
                                     preferred_element_type=jnp.float32))
    s = jnp.concatenate(parts, axis=1)[:, : _NUM_CLASS]
    c = jnp.sum(pcnt_ref[...].reshape(_NW, _CNTP // 128, 128), axis=0)
    w = lax.dot_general(c, sel, (((1,), (0,)), ((), ())),
                        preferred_element_type=jnp.float32)
    cnt = jnp.concatenate([w[0:1, :], w[1:2, :], w[2:3, :]],
                          axis=1)[:, : _NUM_CLASS]
    m = jnp.where(cnt > 0.0, s / jnp.maximum(cnt, 1.0), 0.0)
    n2 = jnp.sum(m * m, axis=0, keepdims=True)
    fn = m / jnp.maximum(jnp.sqrt(n2), 1e-12)
    g = lax.dot_general(fn, fn, (((0,), (0,)), ((), ())),
                        preferred_element_type=jnp.float32)
    d = 1.0 - g
    ii = lax.broadcasted_iota(jnp.int32, (_NUM_CLASS, _NUM_CLASS), 0)
    jj = lax.broadcasted_iota(jnp.int32, (_NUM_CLASS, _NUM_CLASS), 1)
    d = jnp.where(ii == jj, 2.0, d)
    presentf = jnp.where(cnt > 0.0, 1.0, 0.0)
    pair = lax.dot_general(presentf, presentf, (((0,), (0,)), ((), ())),
                           preferred_element_type=jnp.float32)
    vals = pair * jnp.maximum(_MARGIN - d, 0.0)
    n = jnp.sum(presentf)
    out_ref[...] = jnp.reshape(_FACTOR * jnp.sum(vals) / (n * n), (1, 1))


_loss_tc = pl.pallas_call(
    _loss_body,
    out_shape=jax.ShapeDtypeStruct((1, 1), jnp.float32),
)


def kernel(features, labels, prototypes):
    del prototypes
    feat2 = features.reshape(2 * _C * _H, _W)
    lab2 = labels.reshape(2 * _H, _W)
    psum, pcnt = _sc_segsum(feat2, lab2)
    loss = _loss_tc(psum.reshape(_NW * _ACCP // 128, 128),
                    pcnt.reshape(_NW * _CNTP // 128, 128))
    return loss[0, 0]


# speedup vs baseline: 1.4172x; 1.1879x over previous
import functools

import jax
import jax.numpy as jnp
from jax import lax
from jax.experimental import pallas as pl
from jax.experimental.pallas import tpu as pltpu
from jax.experimental.pallas import tpu_sc as plsc

_NUM_CLASS = 19
_MARGIN = 0.5
_FACTOR = 1.0

_NC = 2
_NS = 16
_NW = _NC * _NS
_L = 16

_C = 96
_H = 512
_W = 512
_RPW = _H // _NS
_PPW = _RPW * _W
_KPAD = 512
_ACCP = _C * _KPAD
_CNTP = 1024
_JV = _W // _L


_CG = 4
_NG = _C // _CG
_RH = _RPW // 2


def _sc_body(feat_hbm, lab_hbm, psum_hbm, pcnt_hbm,
             lbuf, fbuf0, fbuf1, fbuf2, fbuf3, acc, cnt,
             sem0, sem1, sem2, sem3):
    wid = lax.axis_index("c") * _NS + lax.axis_index("s")
    b = wid // _NS
    h0 = (wid % _NS) * _RPW

    iota = lax.iota(jnp.int32, _L)
    ones = jnp.ones((_L,), jnp.float32)
    zeros = jnp.zeros((_L,), jnp.float32)

    pltpu.sync_copy(lab_hbm.at[pl.ds(b * _H + h0, _RPW), :], lbuf)

    def _feat_src(c, half):
        row = (b * _C + c) * _H + h0 + half * _RH
        return feat_hbm.at[pl.ds(row, _RH), :]

    pltpu.make_async_copy(_feat_src(0, 0), fbuf0, sem0).start()
    pltpu.make_async_copy(_feat_src(1, 0), fbuf1, sem1).start()
    pltpu.make_async_copy(_feat_src(0, 1), fbuf2, sem2).start()
    pltpu.make_async_copy(_feat_src(1, 1), fbuf3, sem3).start()

    @plsc.parallel_loop(0, _ACCP // _L, unroll=8)
    def _zacc(i):
        acc[pl.ds(i * _L, _L)] = zeros

    @plsc.parallel_loop(0, _CNTP // _L, unroll=4)
    def _zcnt(i):
        cnt[pl.ds(i * _L, _L)] = zeros

    @plsc.parallel_loop(0, _RPW, unroll=2)
    def _cbody(i):
        for j in range(_JV):
            lv = lbuf[i, pl.ds(j * _L, _L)]
            plsc.addupdate_scatter(cnt, [lv * _L + iota], ones)

    pairs = ((fbuf0, sem0, fbuf1, sem1), (fbuf2, sem2, fbuf3, sem3))

    def _chan(sb, carry):
        c0 = sb * 2
        base0 = iota + c0 * _KPAD
        base1 = base0 + _KPAD
        for half, (fba, sema, fbb, semb) in enumerate(pairs):
            pltpu.make_async_copy(_feat_src(0, 0), fba, sema).wait()
            pltpu.make_async_copy(_feat_src(0, 0), fbb, semb).wait()
            hbase = half * _RH

            @plsc.parallel_loop(0, _RH * _JV, unroll=8)
            def _inner(t):
                i = t // _JV
                j = t % _JV
                lv = lbuf[hbase + i, pl.ds(j * _L, _L)] * _L
                v0 = fba[i, pl.ds(j * _L, _L)]
                v1 = fbb[i, pl.ds(j * _L, _L)]
                plsc.addupdate_scatter(acc, [lv + base0], v0)
                plsc.addupdate_scatter(acc, [lv + base1], v1)

            @pl.when(sb < _C // 2 - 1)
            def _prefetch():
                pltpu.make_async_copy(_feat_src(c0 + 2, half), fba, sema).start()
                pltpu.make_async_copy(_feat_src(c0 + 3, half), fbb, semb).start()
        return carry

    lax.fori_loop(0, _C // 2, _chan, 0)

    pltpu.sync_copy(acc, psum_hbm.at[pl.ds(wid * _ACCP, _ACCP)])
    pltpu.sync_copy(cnt, pcnt_hbm.at[pl.ds(wid * _CNTP, _CNTP)])


_sc_segsum = functools.partial(
    pl.kernel,
    out_type=(
        jax.ShapeDtypeStruct((_NW * _ACCP,), jnp.float32),
        jax.ShapeDtypeStruct((_NW * _CNTP,), jnp.float32),
    ),
    mesh=plsc.VectorSubcoreMesh(
        core_axis_name="c", subcore_axis_name="s",
        num_cores=_NC, num_subcores=_NS),
    compiler_params=pltpu.CompilerParams(
        needs_layout_passes=False, use_tc_tiling_on_sc=True),
    scratch_types=[
        pltpu.VMEM((_RPW, _W), jnp.int32),
        pltpu.VMEM((_RH, _W), jnp.float32),
        pltpu.VMEM((_RH, _W), jnp.float32),
        pltpu.VMEM((_RH, _W), jnp.float32),
        pltpu.VMEM((_RH, _W), jnp.float32),
        pltpu.VMEM((_ACCP,), jnp.float32),
        pltpu.VMEM((_CNTP,), jnp.float32),
        pltpu.SemaphoreType.DMA,
        pltpu.SemaphoreType.DMA,
        pltpu.SemaphoreType.DMA,
        pltpu.SemaphoreType.DMA,
    ],
)(_sc_body)


def _loss_body(psum_ref, pcnt_ref, out_ref):
    x = psum_ref[...].reshape(_NW, _ACCP // 128, 128)
    z = jnp.sum(x, axis=0)
    ii = lax.broadcasted_iota(jnp.int32, (128, 8), 0)
    jj = lax.broadcasted_iota(jnp.int32, (128, 8), 1)
    sel = jnp.where(ii // _L == jj, 1.0, 0.0)
    y = lax.dot_general(z, sel, (((1,), (0,)), ((), ())),
                        preferred_element_type=jnp.float32)
    ri = lax.broadcasted_iota(jnp.int32, (_C, 4 * _C), 0)
    rj = lax.broadcasted_iota(jnp.int32, (_C, 4 * _C), 1)
    parts = []
    for a in range(3):
        ga = jnp.where(rj == 4 * ri + a, 1.0, 0.0)
        parts.append(lax.dot_general(ga, y, (((1,), (0,)), ((), ())),
                                     preferred_element_type=jnp.float32))
    s = jnp.concatenate(parts, axis=1)[:, : _NUM_CLASS]
    c = jnp.sum(pcnt_ref[...].reshape(_NW, _CNTP // 128, 128), axis=0)
    w = lax.dot_general(c, sel, (((1,), (0,)), ((), ())),
                        preferred_element_type=jnp.float32)
    cnt = jnp.concatenate([w[0:1, :], w[1:2, :], w[2:3, :]],
                          axis=1)[:, : _NUM_CLASS]
    m = jnp.where(cnt > 0.0, s / jnp.maximum(cnt, 1.0), 0.0)
    n2 = jnp.sum(m * m, axis=0, keepdims=True)
    fn = m / jnp.maximum(jnp.sqrt(n2), 1e-12)
    g = lax.dot_general(fn, fn, (((0,), (0,)), ((), ())),
                        preferred_element_type=jnp.float32)
    d = 1.0 - g
    ii = lax.broadcasted_iota(jnp.int32, (_NUM_CLASS, _NUM_CLASS), 0)
    jj = lax.broadcasted_iota(jnp.int32, (_NUM_CLASS, _NUM_CLASS), 1)
    d = jnp.where(ii == jj, 2.0, d)
    presentf = jnp.where(cnt > 0.0, 1.0, 0.0)
    pair = lax.dot_general(presentf, presentf, (((0,), (0,)), ((), ())),
                           preferred_element_type=jnp.float32)
    vals = pair * jnp.maximum(_MARGIN - d, 0.0)
    n = jnp.sum(presentf)
    out_ref[...] = jnp.reshape(_FACTOR * jnp.sum(vals) / (n * n), (1, 1))


_loss_tc = pl.pallas_call(
    _loss_body,
    out_shape=jax.ShapeDtypeStruct((1, 1), jnp.float32),
)


def kernel(features, labels, prototypes):
    del prototypes
    feat2 = features.reshape(2 * _C * _H, _W)
    lab2 = labels.reshape(2 * _H, _W)
    psum, pcnt = _sc_segsum(feat2, lab2)
    loss = _loss_tc(psum.reshape(_NW * _ACCP // 128, 128),
                    pcnt.reshape(_NW * _CNTP // 128, 128))
    return loss[0, 0]


# speedup vs baseline: 1.4892x; 1.0508x over previous
import functools

import jax
import jax.numpy as jnp
from jax import lax
from jax.experimental import pallas as pl
from jax.experimental.pallas import tpu as pltpu
from jax.experimental.pallas import tpu_sc as plsc

_NUM_CLASS = 19
_MARGIN = 0.5
_FACTOR = 1.0

_NC = 2
_NS = 16
_NW = _NC * _NS
_L = 16

_C = 96
_H = 512
_W = 512
_RPW = _H // _NS
_PPW = _RPW * _W
_KPAD = 512
_ACCP = _C * _KPAD
_CNTP = 1024
_JV = _W // _L


_CG = 4
_NG = _C // _CG
_RH = _RPW // 2
_RQ = _RPW // 4


def _sc_body(feat_hbm, lab_hbm, psum_hbm, pcnt_hbm,
             lbuf, fb0, fb1, fb2, fb3, fb4, fb5, fb6, fb7, acc, cnt,
             sm0, sm1, sm2, sm3, sm4, sm5, sm6, sm7):
    wid = lax.axis_index("c") * _NS + lax.axis_index("s")
    b = wid // _NS
    h0 = (wid % _NS) * _RPW

    iota = lax.iota(jnp.int32, _L)
    ones = jnp.ones((_L,), jnp.float32)
    zeros = jnp.zeros((_L,), jnp.float32)

    pltpu.sync_copy(lab_hbm.at[pl.ds(b * _H + h0, _RPW), :], lbuf)

    def _feat_src(c, q):
        row = (b * _C + c) * _H + h0 + q * _RQ
        return feat_hbm.at[pl.ds(row, _RQ), :]

    quadA = ((fb0, sm0), (fb1, sm1), (fb2, sm2), (fb3, sm3))
    quadB = ((fb4, sm4), (fb5, sm5), (fb6, sm6), (fb7, sm7))

    for k, (fb, sm) in enumerate(quadA):
        pltpu.make_async_copy(_feat_src(k, 0), fb, sm).start()
    for k, (fb, sm) in enumerate(quadB):
        pltpu.make_async_copy(_feat_src(k, 1), fb, sm).start()

    @plsc.parallel_loop(0, _ACCP // _L, unroll=8)
    def _zacc(i):
        acc[pl.ds(i * _L, _L)] = zeros

    @plsc.parallel_loop(0, _CNTP // _L, unroll=4)
    def _zcnt(i):
        cnt[pl.ds(i * _L, _L)] = zeros

    @plsc.parallel_loop(0, _RPW, unroll=2)
    def _cbody(i):
        for j in range(_JV):
            lv = lbuf[i, pl.ds(j * _L, _L)]
            plsc.addupdate_scatter(cnt, [lv * _L + iota], ones)

    def _chan(a, carry):
        c0 = a * 4
        bases = [iota + (c0 + k) * _KPAD for k in range(4)]
        for q in range(4):
            quad = quadA if q % 2 == 0 else quadB
            for fb, sm in quad:
                pltpu.make_async_copy(_feat_src(0, 0), fb, sm).wait()
            hbase = q * _RQ

            @plsc.parallel_loop(0, _RQ * _JV, unroll=8)
            def _inner(t):
                i = t // _JV
                j = t % _JV
                lv = lbuf[hbase + i, pl.ds(j * _L, _L)] * _L
                for k, (fb, sm) in enumerate(quad):
                    v = fb[i, pl.ds(j * _L, _L)]
                    plsc.addupdate_scatter(acc, [lv + bases[k]], v)

            if q < 2:
                for k, (fb, sm) in enumerate(quad):
                    pltpu.make_async_copy(_feat_src(c0 + k, q + 2), fb, sm).start()
            else:
                @pl.when(a < _C // 4 - 1)
                def _prefetch():
                    for k, (fb, sm) in enumerate(quad):
                        pltpu.make_async_copy(
                            _feat_src(c0 + 4 + k, q - 2), fb, sm).start()
        return carry

    lax.fori_loop(0, _C // 4, _chan, 0)

    pltpu.sync_copy(acc, psum_hbm.at[pl.ds(wid * _ACCP, _ACCP)])
    pltpu.sync_copy(cnt, pcnt_hbm.at[pl.ds(wid * _CNTP, _CNTP)])


_sc_segsum = functools.partial(
    pl.kernel,
    out_type=(
        jax.ShapeDtypeStruct((_NW * _ACCP,), jnp.float32),
        jax.ShapeDtypeStruct((_NW * _CNTP,), jnp.float32),
    ),
    mesh=plsc.VectorSubcoreMesh(
        core_axis_name="c", subcore_axis_name="s",
        num_cores=_NC, num_subcores=_NS),
    compiler_params=pltpu.CompilerParams(
        needs_layout_passes=False, use_tc_tiling_on_sc=True),
    scratch_types=[
        pltpu.VMEM((_RPW, _W), jnp.int32),
        pltpu.VMEM((_RQ, _W), jnp.float32),
        pltpu.VMEM((_RQ, _W), jnp.float32),
        pltpu.VMEM((_RQ, _W), jnp.float32),
        pltpu.VMEM((_RQ, _W), jnp.float32),
        pltpu.VMEM((_RQ, _W), jnp.float32),
        pltpu.VMEM((_RQ, _W), jnp.float32),
        pltpu.VMEM((_RQ, _W), jnp.float32),
        pltpu.VMEM((_RQ, _W), jnp.float32),
        pltpu.VMEM((_ACCP,), jnp.float32),
        pltpu.VMEM((_CNTP,), jnp.float32),
        pltpu.SemaphoreType.DMA,
        pltpu.SemaphoreType.DMA,
        pltpu.SemaphoreType.DMA,
        pltpu.SemaphoreType.DMA,
        pltpu.SemaphoreType.DMA,
        pltpu.SemaphoreType.DMA,
        pltpu.SemaphoreType.DMA,
        pltpu.SemaphoreType.DMA,
    ],
)(_sc_body)


def _loss_body(psum_ref, pcnt_ref, out_ref):
    x = psum_ref[...].reshape(_NW, _ACCP // 128, 128)
    z = jnp.sum(x, axis=0)
    ii = lax.broadcasted_iota(jnp.int32, (128, 8), 0)
    jj = lax.broadcasted_iota(jnp.int32, (128, 8), 1)
    sel = jnp.where(ii // _L == jj, 1.0, 0.0)
    y = lax.dot_general(z, sel, (((1,), (0,)), ((), ())),
                        preferred_element_type=jnp.float32)
    ri = lax.broadcasted_iota(jnp.int32, (_C, 4 * _C), 0)
    rj = lax.broadcasted_iota(jnp.int32, (_C, 4 * _C), 1)
    parts = []
    for a in range(3):
        ga = jnp.where(rj == 4 * ri + a, 1.0, 0.0)
        parts.append(lax.dot_general(ga, y, (((1,), (0,)), ((), ())),
                                     preferred_element_type=jnp.float32))
    s = jnp.concatenate(parts, axis=1)[:, : _NUM_CLASS]
    c = jnp.sum(pcnt_ref[...].reshape(_NW, _CNTP // 128, 128), axis=0)
    w = lax.dot_general(c, sel, (((1,), (0,)), ((), ())),
                        preferred_element_type=jnp.float32)
    cnt = jnp.concatenate([w[0:1, :], w[1:2, :], w[2:3, :]],
                          axis=1)[:, : _NUM_CLASS]
    m = jnp.where(cnt > 0.0, s / jnp.maximum(cnt, 1.0), 0.0)
    n2 = jnp.sum(m * m, axis=0, keepdims=True)
    fn = m / jnp.maximum(jnp.sqrt(n2), 1e-12)
    g = lax.dot_general(fn, fn, (((0,), (0,)), ((), ())),
                        preferred_element_type=jnp.float32)
    d = 1.0 - g
    ii = lax.broadcasted_iota(jnp.int32, (_NUM_CLASS, _NUM_CLASS), 0)
    jj = lax.broadcasted_iota(jnp.int32, (_NUM_CLASS, _NUM_CLASS), 1)
    d = jnp.where(ii == jj, 2.0, d)
    presentf = jnp.where(cnt > 0.0, 1.0, 0.0)
    pair = lax.dot_general(presentf, presentf, (((0,), (0,)), ((), ())),
                           preferred_element_type=jnp.float32)
    vals = pair * jnp.maximum(_MARGIN - d, 0.0)
    n = jnp.sum(presentf)
    out_ref[...] = jnp.reshape(_FACTOR * jnp.sum(vals) / (n * n), (1, 1))


_loss_tc = pl.pallas_call(
    _loss_body,
    out_shape=jax.ShapeDtypeStruct((1, 1), jnp.float32),
)


def kernel(features, labels, prototypes):
    del prototypes
    feat2 = features.reshape(2 * _C * _H, _W)
    lab2 = labels.reshape(2 * _H, _W)
    psum, pcnt = _sc_segsum(feat2, lab2)
    loss = _loss_tc(psum.reshape(_NW * _ACCP // 128, 128),
                    pcnt.reshape(_NW * _CNTP // 128, 128))
    return loss[0, 0]
